# baseline (speedup 1.0000x reference)
import numpy as np
import jax
import jax.numpy as jnp
from jax import lax
from jax.experimental import pallas as pl
from jax.experimental.pallas import tpu as pltpu
from jax.experimental.pallas import tpu_sc as plsc

VOCAB = 100000
DIM = 128
BASE = 512
NC, NS = 2, 16
NW = NC * NS
GCH = 128
SEQ_PER_BLK = 64


def _positional_table(seq_len):
    position = np.arange(BASE, dtype=np.float32)[:, None]
    div_term = np.exp(
        np.arange(0, DIM, 2, dtype=np.float32) * (-np.log(10000.0) / DIM))
    pe = np.zeros((BASE, DIM), dtype=np.float32)
    pe[:, 0::2] = np.sin(position * div_term)
    pe[:, 1::2] = np.cos(position * div_term)
    return jnp.asarray(pe[:seq_len])


def _minmax_body(w_ref, mn_ref, mx_ref):
    i = pl.program_id(0)
    bmn = jnp.min(w_ref[...])
    bmx = jnp.max(w_ref[...])

    @pl.when(i == 0)
    def _():
        mn_ref[0, 0] = bmn
        mx_ref[0, 0] = bmx

    @pl.when(i != 0)
    def _():
        mn_ref[0, 0] = jnp.minimum(mn_ref[0, 0], bmn)
        mx_ref[0, 0] = jnp.maximum(mx_ref[0, 0], bmx)


def _weight_minmax(weight):
    rows = 2000
    grid = VOCAB // rows
    mn, mx = pl.pallas_call(
        _minmax_body,
        grid=(grid,),
        in_specs=[pl.BlockSpec((rows, DIM), lambda i: (i, 0))],
        out_specs=[
            pl.BlockSpec((1, 1), lambda i: (0, 0), memory_space=pltpu.SMEM),
            pl.BlockSpec((1, 1), lambda i: (0, 0), memory_space=pltpu.SMEM),
        ],
        out_shape=[
            jax.ShapeDtypeStruct((1, 1), jnp.float32),
            jax.ShapeDtypeStruct((1, 1), jnp.float32),
        ],
    )(weight)
    return mn[0, 0], mx[0, 0]


def _make_sc_gather(tokens):
    tok_w = tokens // NW
    nchunk = tok_w // GCH
    nbuf = 4
    assert tokens % NW == 0 and tok_w % GCH == 0 and nchunk >= nbuf

    mesh = plsc.VectorSubcoreMesh(core_axis_name="c", subcore_axis_name="s")

    def body(ids_hbm, w_hbm, emb_hbm,
             idv, rows0, rows1, rows2, rows3,
             gsem0, gsem1, gsem2, gsem3, ssem0, ssem1, ssem2, ssem3):
        rows = (rows0, rows1, rows2, rows3)
        gsems = (gsem0, gsem1, gsem2, gsem3)
        ssems = (ssem0, ssem1, ssem2, ssem3)

        cid = lax.axis_index("c")
        sid = lax.axis_index("s")
        wid = sid * NC + cid
        tok0 = wid * tok_w

        pltpu.sync_copy(ids_hbm.at[pl.ds(tok0, tok_w)], idv)

        def start_gather(c, rowsb, gsem):
            pltpu.async_copy(w_hbm.at[idv.at[pl.ds(c * GCH, GCH)]],
                             rowsb, gsem)

        def wait_gather(c, rowsb, gsem):
            pltpu.make_async_copy(w_hbm.at[idv.at[pl.ds(c * GCH, GCH)]],
                                  rowsb, gsem).wait()

        def start_scatter(c, rowsb, ssem):
            t0 = tok0 + c * GCH
            pltpu.async_copy(rowsb, emb_hbm.at[pl.ds(t0, GCH)], ssem)

        def wait_scatter(c, rowsb, ssem):
            t0 = tok0 + c * GCH
            pltpu.make_async_copy(rowsb, emb_hbm.at[pl.ds(t0, GCH)],
                                  ssem).wait()

        def step(c, b):
            wait_gather(c, rows[b], gsems[b])
            start_scatter(c, rows[b], ssems[b])
            g = c + nbuf - 1
            b2 = (b + nbuf - 1) % nbuf
            if isinstance(g, int) and g >= nchunk:
                return

            def issue():
                start_gather(g, rows[b2], gsems[b2])

            if isinstance(g, int):
                if g >= nbuf:
                    wait_scatter(g - nbuf, rows[b2], ssems[b2])
                issue()
            else:
                @pl.when(g >= nbuf)
                def _():
                    wait_scatter(g - nbuf, rows[b2], ssems[b2])
                issue()

        for c in range(nbuf - 1):
            start_gather(c, rows[c], gsems[c])

        nsteady = (nchunk - (nbuf - 1)) // nbuf
        def outer(k, carry):
            c0 = nbuf * k
            for b in range(nbuf):
                step(c0 + b, b)
            return carry
        lax.fori_loop(0, nsteady, outer, 0)
        for c in range(nsteady * nbuf, nchunk):
            step(c, c % nbuf)
        for c in range(nchunk - nbuf, nchunk):
            wait_scatter(c, rows[c % nbuf], ssems[c % nbuf])

    return pl.kernel(
        body,
        out_type=jax.ShapeDtypeStruct((tokens, DIM), jnp.float32),
        mesh=mesh,
        scratch_types=[
            pltpu.VMEM((tok_w,), jnp.int32),
            pltpu.VMEM((GCH, DIM), jnp.float32),
            pltpu.VMEM((GCH, DIM), jnp.float32),
            pltpu.VMEM((GCH, DIM), jnp.float32),
            pltpu.VMEM((GCH, DIM), jnp.float32),
            pltpu.SemaphoreType.DMA,
            pltpu.SemaphoreType.DMA,
            pltpu.SemaphoreType.DMA,
            pltpu.SemaphoreType.DMA,
            pltpu.SemaphoreType.DMA,
            pltpu.SemaphoreType.DMA,
            pltpu.SemaphoreType.DMA,
            pltpu.SemaphoreType.DMA,
        ],
    )


def _ln_body(emb_ref, pe_ref, cst_ref, gam_ref, bet_ref, out_ref):
    inv = cst_ref[0, 0]
    zp = cst_ref[0, 1]
    scale = cst_ref[0, 2]
    x = emb_ref[...]
    q = jnp.round(x * inv + zp)
    e = q * scale + pe_ref[...]
    mean = jnp.mean(e, axis=-1, keepdims=True)
    var = jnp.mean(e * e, axis=-1, keepdims=True) - mean * mean
    r = lax.rsqrt(var + 1e-5)
    out_ref[...] = (e - mean) * r * gam_ref[...] + bet_ref[...]


def _ln_pass(emb3, pe3, cst, gamma, beta, batch, seq):
    grid = batch // SEQ_PER_BLK
    assert batch % SEQ_PER_BLK == 0
    return pl.pallas_call(
        _ln_body,
        grid=(grid,),
        in_specs=[
            pl.BlockSpec((SEQ_PER_BLK, seq, DIM), lambda i: (i, 0, 0)),
            pl.BlockSpec((1, seq, DIM), lambda i: (0, 0, 0)),
            pl.BlockSpec((1, 3), lambda i: (0, 0), memory_space=pltpu.SMEM),
            pl.BlockSpec((1, 1, DIM), lambda i: (0, 0, 0)),
            pl.BlockSpec((1, 1, DIM), lambda i: (0, 0, 0)),
        ],
        out_specs=pl.BlockSpec((SEQ_PER_BLK, seq, DIM), lambda i: (i, 0, 0)),
        out_shape=jax.ShapeDtypeStruct((batch, seq, DIM), jnp.float32),
    )(emb3, pe3, cst, gamma, beta)


def kernel(input_ids, weight, gamma, beta):
    batch, seq = input_ids.shape
    tokens = batch * seq
    pe = _positional_table(seq)

    wmin, wmax = _weight_minmax(weight)
    scale = (wmax - wmin) / 255.0
    zp = -128.0 - wmin / scale
    cst = jnp.stack([1.0 / scale, zp, scale]).reshape(1, 3)
    pe3 = (pe - zp * scale).reshape(1, seq, DIM)

    ids_flat = input_ids.reshape(tokens).astype(jnp.int32)
    gather = _make_sc_gather(tokens)
    emb = gather(ids_flat, weight)
    return _ln_pass(emb.reshape(batch, seq, DIM), pe3, cst,
                    gamma.reshape(1, 1, DIM), beta.reshape(1, 1, DIM),
                    batch, seq)
